# baseline (speedup 1.0000x reference)
import functools
import math

import jax
import jax.numpy as jnp
from jax import lax
from jax.experimental import pallas as pl
from jax.experimental.pallas import tpu as pltpu
from jax.experimental.pallas import tpu_sc as plsc

NCLS = 80
NANCH = 3
NLVL = 3
BAL = (4.0, 1.0, 0.4)
GBIAS = 0.5
EPAD = 4608

_FEAT = ((80, 80), (40, 40), (20, 20))

_ATAN_C = (0.9999999937488345, -0.33333137929908097, 0.19993693394198278,
           -0.14211098330283195, 0.10667454712913349, -0.07556827050084194,
           0.04327732083475509, -0.01641258775269415, 0.002932602096126738)


def _atan_pos(z):
    inv = z > 1.0
    x = jnp.where(inv, 1.0 / z, z)
    x2 = x * x
    q = jnp.full_like(x, _ATAN_C[-1])
    for c in _ATAN_C[-2::-1]:
        q = q * x2 + c
    a = x * q
    return jnp.where(inv, math.pi / 2 - a, a)
_M = tuple(16 * 3 * h * w for (w, h) in _FEAT)
_KTILE = (24, 24, 16)
_VROWS = (204000, 51000, 13600)
_SEG = tuple(r // (85 * k) for r, k in zip(_VROWS, _KTILE))


def _prep_body(tT_ref, anch_ref, pl_ref, idx_ref):
    nt = tT_ref.shape[1]
    tb = tT_ref[0:1, :]
    tcl = tT_ref[1:2, :]
    txn = tT_ref[2:3, :]
    tyn = tT_ref[3:4, :]
    twn = tT_ref[4:5, :]
    thn = tT_ref[5:6, :]
    row = lax.broadcasted_iota(jnp.int32, (15, 1), 0)
    o = row // 3
    a = row % 3
    offx = jnp.where(o == 1, GBIAS, jnp.where(o == 3, -GBIAS, 0.0))
    offy = jnp.where(o == 2, GBIAS, jnp.where(o == 4, -GBIAS, 0.0))
    zf = jnp.zeros((15, nt), jnp.float32)
    clsf = tcl.astype(jnp.int32).astype(jnp.float32)
    for l in range(NLVL):
        gw, gh = _FEAT[l]
        gx = txn * gw
        gy = tyn * gh
        w = twn * gw
        h = thn * gh
        aw = anch_ref[l, :, 0:1]
        ah = anch_ref[l, :, 1:2]
        rw = w / aw
        rh = h / ah
        rmax = jnp.maximum(jnp.maximum(rw, 1.0 / rw), jnp.maximum(rh, 1.0 / rh))
        fm = rmax < 4.0
        jm = ((gx - jnp.floor(gx)) < GBIAS) & (gx > 1.0)
        km = ((gy - jnp.floor(gy)) < GBIAS) & (gy > 1.0)
        gxi = gw - gx
        gyi = gh - gy
        lmm = ((gxi - jnp.floor(gxi)) < GBIAS) & (gxi > 1.0)
        mm = ((gyi - jnp.floor(gyi)) < GBIAS) & (gyi > 1.0)
        jb = ((o == 0) | ((o == 1) & jm) | ((o == 2) & km)
              | ((o == 3) & lmm) | ((o == 4) & mm))
        mask = jb & fm
        gijx = (gx - offx).astype(jnp.int32)
        gijy = (gy - offy).astype(jnp.int32)
        gi = jnp.clip(gijx, 0, gw - 1)
        gj = jnp.clip(gijy, 0, gh - 1)
        b = tb.astype(jnp.int32)
        idx = ((b * NANCH + a) * gh + gj) * gw + gi
        pl_ref[l, 0] = gx - gi.astype(jnp.float32)
        pl_ref[l, 1] = gy - gj.astype(jnp.float32)
        pl_ref[l, 2] = w + zf
        pl_ref[l, 3] = h + zf
        pl_ref[l, 4] = aw + zf
        pl_ref[l, 5] = ah + zf
        pl_ref[l, 6] = mask.astype(jnp.float32)
        pl_ref[l, 7] = clsf + zf
        idx_ref[l] = jnp.clip(idx, 0, _M[l] - 1)


def _run_prep(tT, anch):
    nt = tT.shape[1]
    return pl.pallas_call(
        _prep_body,
        out_shape=[
            jax.ShapeDtypeStruct((NLVL, 8, 15, nt), jnp.float32),
            jax.ShapeDtypeStruct((NLVL, 15, nt), jnp.int32),
        ],
    )(tT, anch)


def _prep_targets(targets, mapped_anchors):
    nt = targets.shape[0]
    ai = jnp.tile(jnp.arange(NANCH, dtype=jnp.float32).reshape(NANCH, 1), (1, nt))
    t_all = jnp.concatenate(
        [jnp.tile(targets[None], (NANCH, 1, 1)), ai[..., None]], axis=-1)
    off = jnp.array([[0, 0], [1, 0], [0, 1], [-1, 0], [0, -1]],
                    dtype=jnp.float32) * GBIAS
    out = []
    for i in range(NLVL):
        anchors = mapped_anchors[i]
        gw, gh = _FEAT[i]
        gain = jnp.array([1.0, 1.0, gw, gh, gw, gh, 1.0], dtype=jnp.float32)
        t = t_all * gain
        r = t[..., 4:6] / anchors[:, None]
        fmask = jnp.max(jnp.maximum(r, 1.0 / r), axis=2) < 4.0
        t = t.reshape(NANCH * nt, 7)
        vmask = fmask.reshape(NANCH * nt)
        gxy = t[:, 2:4]
        gxi = jnp.array([gw, gh], dtype=jnp.float32) - gxy
        jk = (gxy % 1 < GBIAS) & (gxy > 1)
        lm = (gxi % 1 < GBIAS) & (gxi > 1)
        jm, km = jk[:, 0], jk[:, 1]
        lmm, mm = lm[:, 0], lm[:, 1]
        jmask = jnp.stack([jnp.ones_like(jm), jm, km, lmm, mm])
        mask = (jmask & vmask[None]).reshape(5 * NANCH * nt)
        t = jnp.tile(t[None], (5, 1, 1)).reshape(5 * NANCH * nt, 7)
        offsets = (jnp.zeros_like(gxy)[None] + off[:, None]).reshape(
            5 * NANCH * nt, 2)
        bc = t[:, :2]
        gxy2 = t[:, 2:4]
        gwh = t[:, 4:6]
        aidx = t[:, 6].astype(jnp.int32)
        b = bc[:, 0].astype(jnp.int32)
        cls = bc[:, 1].astype(jnp.int32)
        gij = (gxy2 - offsets).astype(jnp.int32)
        gi = jnp.clip(gij[:, 0], 0, gw - 1)
        gj = jnp.clip(gij[:, 1], 0, gh - 1)
        bbox = jnp.concatenate(
            [gxy2 - jnp.stack([gi, gj], axis=1).astype(jnp.float32), gwh], axis=1)
        anchor = anchors[aidx]
        idx = ((b * NANCH + aidx) * gh + gj) * gw + gi
        idx = jnp.clip(idx, 0, _M[i] - 1)
        out.append((idx, bbox, anchor, cls, mask))
    return out


_NWORK = 32
_WROWS = EPAD // _NWORK
_CHUNK = 48
_NCH = _WROWS // _CHUNK


_WPW = EPAD * 85 // _NWORK
_GCH = 120
_INNER = 17
_OUTER = _WPW // (_GCH * _INNER)


def _sc_gather_body(p0_hbm, p1_hbm, p2_hbm, w0_hbm, w1_hbm, w2_hbm,
                    g0_hbm, g1_hbm, g2_hbm, idx_v, rows_v, sem):
    wid = lax.axis_index("s") * 2 + lax.axis_index("c")
    for lvl, (p_hbm, w_hbm, g_hbm) in enumerate(
            ((p0_hbm, w0_hbm, g0_hbm), (p1_hbm, w1_hbm, g1_hbm),
             (p2_hbm, w2_hbm, g2_hbm))):
        wbase = wid * _WPW
        pltpu.sync_copy(w_hbm.at[pl.ds(wbase, _WPW)], idx_v)
        pltpu.async_copy(p_hbm.at[idx_v], rows_v, sem).wait()
        pltpu.sync_copy(rows_v, g_hbm.at[pl.ds(wbase, _WPW)])


def _run_sc_gather(p0f, p1f, p2f, w0, w1, w2):
    f32 = jnp.float32
    fn = functools.partial(
        pl.kernel,
        out_type=[
            jax.ShapeDtypeStruct((EPAD * 85,), f32),
            jax.ShapeDtypeStruct((EPAD * 85,), f32),
            jax.ShapeDtypeStruct((EPAD * 85,), f32),
        ],
        mesh=plsc.VectorSubcoreMesh(core_axis_name="c", subcore_axis_name="s"),
        scratch_types=[
            pltpu.VMEM((_WPW,), jnp.int32),
            pltpu.VMEM((_WPW,), f32),
            pltpu.SemaphoreType.DMA,
        ],
    )(_sc_gather_body)
    g0, g1, g2 = fn(p0f, p1f, p2f, w0, w1, w2)
    return (g0.reshape(EPAD, 85), g1.reshape(EPAD, 85),
            g2.reshape(EPAD, 85))


def _pad_to(x, n, val=0):
    return jnp.pad(x, [(0, n - x.shape[0])] + [(0, 0)] * (x.ndim - 1),
                   constant_values=val)


def _main_body(p0_ref, p1_ref, p2_ref, cmask_ref, planes_ref, maskc_ref,
               clsc_ref, g0_ref, g1_ref, g2_ref, o_ref, acc_ref):
    i = pl.program_id(0)

    @pl.when(i == 0)
    def _init():
        for k in range(NLVL):
            acc_ref[k] = 0.0

    bounds = []
    s = 0
    for k in range(NLVL):
        bounds.append((s, s + _SEG[k]))
        s += _SEG[k]

    for k, ref in enumerate((p0_ref, p1_ref, p2_ref)):
        lo, hi = bounds[k]
        kt = _KTILE[k]

        @pl.when((i >= lo) & (i < hi))
        def _dense(ref=ref, k=k, kt=kt):
            m = cmask_ref[...]
            rows = jnp.concatenate(
                [jnp.sum(ref[85 * t:85 * (t + 1), :] * m, axis=0,
                         keepdims=True) for t in range(kt)], axis=0)
            sp = jnp.maximum(rows, 0.0) + jnp.log1p(jnp.exp(-jnp.abs(rows)))
            acc_ref[k] += jnp.sum(sp)

    @pl.when(i == s - 1)
    def _finalize():
        box_tot = 0.0
        conf_tot = 0.0
        cls_tot = 0.0
        eps = 1e-07
        for l, g_ref in enumerate((g0_ref, g1_ref, g2_ref)):
            bx = planes_ref[l, 0]
            by = planes_ref[l, 1]
            bw = planes_ref[l, 2]
            bh = planes_ref[l, 3]
            aw = planes_ref[l, 4]
            ah = planes_ref[l, 5]
            mk = planes_ref[l, 6]
            px = g_ref[:, 0:1].reshape(36, 128)
            py = g_ref[:, 1:2].reshape(36, 128)
            pw = g_ref[:, 2:3].reshape(36, 128)
            ph = g_ref[:, 3:4].reshape(36, 128)
            x4 = g_ref[:, 4:5].reshape(36, 128)
            sig = lambda z: 1.0 / (1.0 + jnp.exp(-z))
            pxv = sig(px) * 2.0 - 0.5
            pyv = sig(py) * 2.0 - 0.5
            pwv = (sig(pw) * 2.0) ** 2 * aw
            phv = (sig(ph) * 2.0) ** 2 * ah
            b1x1, b1x2 = pxv - pwv / 2, pxv + pwv / 2
            b1y1, b1y2 = pyv - phv / 2, pyv + phv / 2
            b2x1, b2x2 = bx - bw / 2, bx + bw / 2
            b2y1, b2y2 = by - bh / 2, by + bh / 2
            inter = (jnp.clip(jnp.minimum(b1x2, b2x2) - jnp.maximum(b1x1, b2x1),
                              0.0, None)
                     * jnp.clip(jnp.minimum(b1y2, b2y2) - jnp.maximum(b1y1, b2y1),
                                0.0, None))
            union = pwv * phv + bw * bh - inter + eps
            iou = inter / union
            cw = jnp.maximum(b1x2, b2x2) - jnp.minimum(b1x1, b2x1)
            ch = jnp.maximum(b1y2, b2y2) - jnp.minimum(b1y1, b2y1)
            c2 = cw ** 2 + ch ** 2 + eps
            rho2 = ((b2x1 + b2x2 - b1x1 - b1x2) ** 2
                    + (b2y1 + b2y2 - b1y1 - b1y2) ** 2) / 4.0
            v = 4.0 / math.pi ** 2 * (_atan_pos(bw / bh)
                                      - _atan_pos(pwv / phv)) ** 2
            alpha = v / (v - iou + (1.0 + eps))
            iou_c = iou - (rho2 / c2 + v * alpha)

            denom = jnp.maximum(jnp.sum(mk), 1.0)
            box_tot += jnp.sum((1.0 - iou_c) * mk) / denom
            iou_d = jnp.clip(iou_c, 0.0, None)
            sub = jnp.sum(mk * x4 * iou_d)
            conf_tot += (acc_ref[l] - sub) / float(_M[l]) * BAL[l]

            pcls = g_ref[:, 5:5 + NCLS]
            mc = maskc_ref[l]
            cc = clsc_ref[l]
            onehot = (lax.broadcasted_iota(jnp.int32, (EPAD, NCLS), 1)
                      .astype(jnp.float32) == cc).astype(jnp.float32)
            closs = (jnp.maximum(pcls, 0.0) - pcls * onehot
                     + jnp.log1p(jnp.exp(-jnp.abs(pcls))))
            cls_tot += jnp.sum(closs * mc) / (denom * NCLS)

        total = (box_tot * 0.05 + conf_tot * 1.0 + cls_tot * 0.5) * 16.0
        o_ref[...] = jnp.broadcast_to(total, (1, 1))


def _run_main(p0v, p1v, p2v, cmask, planes, maskc, clsc, g0, g1, g2):
    nsteps = sum(_SEG)
    b0, b1 = _SEG[0], _SEG[0] + _SEG[1]
    in_specs = [
            pl.BlockSpec((85 * _KTILE[0], 128),
                         lambda i: (jnp.minimum(i, b0 - 1), 0)),
            pl.BlockSpec((85 * _KTILE[1], 128),
                         lambda i: (jnp.clip(i - b0, 0, _SEG[1] - 1), 0)),
            pl.BlockSpec((85 * _KTILE[2], 128),
                         lambda i: (jnp.clip(i - b1, 0, _SEG[2] - 1), 0)),
            pl.BlockSpec((85, 128), lambda i: (0, 0)),
            pl.BlockSpec((NLVL, 8, 36, 128), lambda i: (0, 0, 0, 0)),
            pl.BlockSpec((NLVL, EPAD, 1), lambda i: (0, 0, 0)),
            pl.BlockSpec((NLVL, EPAD, 1), lambda i: (0, 0, 0)),
            pl.BlockSpec((EPAD, 85), lambda i: (0, 0)),
            pl.BlockSpec((EPAD, 85), lambda i: (0, 0)),
            pl.BlockSpec((EPAD, 85), lambda i: (0, 0)),
        ]
    return pl.pallas_call(
        _main_body,
        grid=(nsteps,),
        in_specs=in_specs,
        out_specs=pl.BlockSpec((1, 1), lambda i: (0, 0)),
        out_shape=jax.ShapeDtypeStruct((1, 1), jnp.float32),
        scratch_shapes=[pltpu.SMEM((NLVL,), jnp.float32)],
    )(p0v, p1v, p2v, cmask, planes, maskc, clsc, g0, g1, g2)


def kernel(p0, p1, p2, targets, mapped_anchors):
    prs = [p.reshape(-1, 85) for p in (p0, p1, p2)]
    nt = targets.shape[0]
    ne = 5 * NANCH * nt
    tT = targets.T
    anch = jnp.tile(mapped_anchors, (1, 5, 1))
    pp, idxo = _run_prep(tT, anch)

    padc = jnp.concatenate(
        [jnp.ones((NLVL, 6, EPAD - ne), jnp.float32),
         jnp.zeros((NLVL, 2, EPAD - ne), jnp.float32)], axis=1)
    planes = jnp.concatenate([pp.reshape(NLVL, 8, ne), padc],
                             axis=-1).reshape(NLVL, 8, 36, 128)
    maskc = planes[:, 6].reshape(NLVL, EPAD, 1)
    clsc = planes[:, 7].reshape(NLVL, EPAD, 1)
    idxp = jnp.pad(idxo.reshape(NLVL, ne), ((0, 0), (0, EPAD - ne)))
    warr = [(idxp[i][:, None] * 85
             + jnp.arange(85, dtype=jnp.int32)[None]).reshape(-1)
            for i in range(NLVL)]

    g0, g1, g2 = _run_sc_gather(prs[0].reshape(-1), prs[1].reshape(-1),
                                prs[2].reshape(-1), *warr)
    pviews = [prs[0].reshape(-1, 128), prs[1].reshape(-1, 128),
              jnp.pad(prs[2].reshape(-1), (0, _VROWS[2] * 128 - _M[2] * 85),
                      constant_values=-1e30).reshape(-1, 128)]
    cmask = ((128 * jnp.arange(85, dtype=jnp.int32)[:, None]
              + jnp.arange(128, dtype=jnp.int32)[None]) % 85
             == 4).astype(jnp.float32)
    out = _run_main(pviews[0], pviews[1], pviews[2], cmask,
                    planes, maskc, clsc, g0, g1, g2)
    return out.reshape(1)
